# baseline (speedup 1.0000x reference)
import functools

import jax
import jax.numpy as jnp
from jax import lax
from jax.experimental import pallas as pl
from jax.experimental.pallas import tpu as pltpu
from jax.experimental.pallas import tpu_sc as plsc

_N, _E, _D, _DE, _H, _M = 10000, 320000, 128, 16, 128, 512
_E2 = _E // 2

_NC, _NS = 2, 16
_NW = _NC * _NS
_NP = 10240
_RPS = _NP // _NS
_NBUF = 5
_CH = 40
_EPW = _E2 // _NW
_NCH = _EPW // _CH
_NGRP = _NCH // _NBUF
_WCH = _RPS // _CH

_EB = 2000
_EG = _E2 // _EB
_NB = 1000
_NG = _N // _NB

_F32 = jnp.float32


def _sc_gather_body(table_hbm, idx3_hbm, out_hbm, idxv, rows, *sems):
    c = lax.axis_index("c")
    s = lax.axis_index("s")
    w = s * _NC + c
    base0 = w * _EPW
    pltpu.sync_copy(idx3_hbm.at[w], idxv)

    def issue_gather(k, b):
        pltpu.async_copy(table_hbm.at[idxv.at[k]], rows.at[b], sems[b])

    def wait_rowbuf(b):
        pltpu.make_async_copy(out_hbm.at[pl.ds(0, _CH)], rows.at[b],
                              sems[b]).wait()

    def wait_store(b):
        pltpu.make_async_copy(rows.at[b], out_hbm.at[pl.ds(0, _CH)],
                              sems[b]).wait()

    for b in range(_NBUF):
        issue_gather(b, b)

    def group(g, carry):
        for b in range(_NBUF):
            k = g * _NBUF + b
            wait_rowbuf(b)
            pltpu.async_copy(rows.at[b],
                             out_hbm.at[pl.ds(base0 + k * _CH, _CH)],
                             sems[b])

            @pl.when(g < _NGRP - 1)
            def _next(b=b, k=k):
                wait_store(b)
                issue_gather(k + _NBUF, b)

        return carry

    lax.fori_loop(0, _NGRP, group, 0)
    for b in range(_NBUF):
        wait_store(b)


_sc_gather = pl.kernel(
    _sc_gather_body,
    out_type=jax.ShapeDtypeStruct((_E2, _H), _F32),
    mesh=plsc.VectorSubcoreMesh(core_axis_name="c", subcore_axis_name="s"),
    scratch_types=[
        pltpu.VMEM((_NCH, _CH), jnp.int32),
        pltpu.VMEM((_NBUF, _CH, _H), _F32),
    ] + [pltpu.SemaphoreType.DMA] * _NBUF,
)


def _sc_scatter_body(rows_hbm, idx3_hbm, zeros_hbm, iota2_hbm, part_hbm,
                     acc, idxv, iotav, rows, *sems):
    c = lax.axis_index("c")
    s = lax.axis_index("s")
    w = s * _NC + c
    pltpu.sync_copy(idx3_hbm.at[w], idxv)
    pltpu.sync_copy(iota2_hbm.at[pl.ds(s * _WCH, _WCH)], iotav)

    pltpu.sync_copy(zeros_hbm, rows.at[0])
    zd = [pltpu.async_copy(rows.at[0], acc.at[iotav.at[j]], sems[0])
          for j in range(_WCH)]
    for d in zd:
        d.wait()
    plsc.subcore_barrier()

    base0 = w * _EPW

    def issue_load(k, b):
        pltpu.async_copy(rows_hbm.at[pl.ds(base0 + k * _CH, _CH)],
                         rows.at[b], sems[b])

    def wait_rowbuf(b):
        pltpu.make_async_copy(rows_hbm.at[pl.ds(0, _CH)], rows.at[b],
                              sems[b]).wait()

    def wait_add(b):
        pltpu.make_async_copy(rows.at[b], acc.at[iotav.at[0]],
                              sems[b]).wait()

    for b in range(_NBUF):
        issue_load(b, b)

    def group(g, carry):
        for b in range(_NBUF):
            k = g * _NBUF + b
            wait_rowbuf(b)
            pltpu.async_copy(rows.at[b], acc.at[idxv.at[k]], sems[b],
                             add=True)

            @pl.when(g < _NGRP - 1)
            def _next(b=b, k=k):
                wait_add(b)
                issue_load(k + _NBUF, b)

        return carry

    lax.fori_loop(0, _NGRP, group, 0)
    for b in range(_NBUF):
        wait_add(b)
    plsc.subcore_barrier()

    descs = {}
    for j in range(_WCH):
        b = j % _NBUF
        if j >= _NBUF:
            descs[j - _NBUF].wait()
        pltpu.async_copy(acc.at[iotav.at[j]], rows.at[b], sems[b]).wait()
        descs[j] = pltpu.async_copy(
            rows.at[b], part_hbm.at[c, pl.ds(s * _RPS + j * _CH, _CH)],
            sems[b])
    for j in range(_WCH - _NBUF, _WCH):
        descs[j].wait()


_sc_scatter = pl.kernel(
    _sc_scatter_body,
    out_type=jax.ShapeDtypeStruct((_NC, _NP, _H), _F32),
    mesh=plsc.VectorSubcoreMesh(core_axis_name="c", subcore_axis_name="s"),
    scratch_types=[
        pltpu.VMEM_SHARED((_NP, _H), _F32),
        pltpu.VMEM((_NCH, _CH), jnp.int32),
        pltpu.VMEM((_WCH, _CH), jnp.int32),
        pltpu.VMEM((_NBUF, _CH, _H), _F32),
    ] + [pltpu.SemaphoreType.DMA] * _NBUF,
)


def _mm_body(x_ref, w_ref, o_ref):
    o_ref[...] = jnp.dot(x_ref[...], w_ref[...], preferred_element_type=_F32)


def _tc_matmul(x, w, block):
    n = x.shape[0]
    return pl.pallas_call(
        _mm_body,
        grid=(n // block,),
        in_specs=[
            pl.BlockSpec((block, x.shape[1]), lambda i: (i, 0)),
            pl.BlockSpec(w.shape, lambda i: (0, 0)),
        ],
        out_specs=pl.BlockSpec((block, w.shape[1]), lambda i: (i, 0)),
        out_shape=jax.ShapeDtypeStruct((n, w.shape[1]), _F32),
    )(x, w)


def _merge_body(pa_ref, pb_ref, o_ref):
    o_ref[...] = (pa_ref[0] + pa_ref[1]) + (pb_ref[0] + pb_ref[1])


def _tc_merge(part_a, part_b):
    return pl.pallas_call(
        _merge_body,
        grid=(_NG,),
        in_specs=[pl.BlockSpec((_NC, _NB, _H), lambda i: (0, i, 0))] * 2,
        out_specs=pl.BlockSpec((_NB, _H), lambda i: (i, 0)),
        out_shape=jax.ShapeDtypeStruct((_N, _H), _F32),
    )(part_a, part_b)


def _stage1_body(ps_ref, ea_ref, wie_ref, wh_ref, o_ref):
    inp = jnp.maximum(
        ps_ref[...] + jnp.dot(ea_ref[...], wie_ref[...],
                              preferred_element_type=_F32), 0.0)
    o_ref[...] = jnp.dot(inp, wh_ref[...], preferred_element_type=_F32)


def _tc_stage1(ps, ea, w_ie, w_h):
    return pl.pallas_call(
        _stage1_body,
        grid=(_EG,),
        in_specs=[
            pl.BlockSpec((_EB, _H), lambda i: (i, 0)),
            pl.BlockSpec((_EB, _DE), lambda i: (i, 0)),
            pl.BlockSpec((_DE, _H), lambda i: (0, 0)),
            pl.BlockSpec((_H, _H), lambda i: (0, 0)),
        ],
        out_specs=pl.BlockSpec((_EB, _H), lambda i: (i, 0)),
        out_shape=jax.ShapeDtypeStruct((_E2, _H), _F32),
    )(ps, ea, w_ie, w_h)


def _pair_swap(g):
    up = jnp.concatenate([g[1:], g[:1]], axis=0)
    dn = jnp.concatenate([g[-1:], g[:-1]], axis=0)
    par = (lax.broadcasted_iota(jnp.int32, g.shape, 0) % 2) == 0
    return jnp.where(par, up, dn)


def _iter_body(ps_ref, ea_ref, s_ref, g_ref, wie_ref, wh_ref, o_ref, *, last):
    inp = jnp.maximum(
        ps_ref[...] + jnp.dot(ea_ref[...], wie_ref[...],
                              preferred_element_type=_F32), 0.0)
    h = jnp.maximum(inp + s_ref[...] - _pair_swap(g_ref[...]), 0.0)
    if last:
        o_ref[...] = h
    else:
        o_ref[...] = jnp.dot(h, wh_ref[...], preferred_element_type=_F32)


def _tc_iter(ps, ea, s, g, w_ie, w_h, last):
    return pl.pallas_call(
        functools.partial(_iter_body, last=last),
        grid=(_EG,),
        in_specs=[
            pl.BlockSpec((_EB, _H), lambda i: (i, 0)),
            pl.BlockSpec((_EB, _DE), lambda i: (i, 0)),
            pl.BlockSpec((_EB, _H), lambda i: (i, 0)),
            pl.BlockSpec((_EB, _H), lambda i: (i, 0)),
            pl.BlockSpec((_DE, _H), lambda i: (0, 0)),
            pl.BlockSpec((_H, _H), lambda i: (0, 0)),
        ],
        out_specs=pl.BlockSpec((_EB, _H), lambda i: (i, 0)),
        out_shape=jax.ShapeDtypeStruct((_E2, _H), _F32),
    )(ps, ea, s, g, w_ie, w_h)


def _final_body(x_ref, apa_ref, apb_ref, mol_ref, wox_ref, woh_ref, wffn_ref,
                o_ref, sums, cnts):
    i = pl.program_id(0)

    @pl.when(i == 0)
    def _zero():
        sums[...] = jnp.zeros_like(sums)
        cnts[...] = jnp.zeros_like(cnts)

    a = (apa_ref[0] + apa_ref[1]) + (apb_ref[0] + apb_ref[1])
    atom = jnp.maximum(
        jnp.dot(x_ref[...], wox_ref[...], preferred_element_type=_F32)
        + jnp.dot(a, woh_ref[...], preferred_element_type=_F32), 0.0)
    mol = mol_ref[0]
    mrow = lax.broadcasted_iota(jnp.int32, (_M, _NB), 0)
    oh = (mrow == mol).astype(_F32)
    sums[...] += jnp.dot(oh, atom, preferred_element_type=_F32)
    cnts[...] += jnp.dot(oh, jnp.ones_like(atom), preferred_element_type=_F32)

    @pl.when(i == pl.num_programs(0) - 1)
    def _emit():
        mv = sums[...] / jnp.maximum(cnts[...], 1.0)
        o_ref[...] = jnp.maximum(
            jnp.dot(mv, wffn_ref[...], preferred_element_type=_F32), 0.0)


def _tc_final(x, apart_a, apart_b, mol3, w_ox, w_oh, w_ffn):
    return pl.pallas_call(
        _final_body,
        grid=(_NG,),
        in_specs=[
            pl.BlockSpec((_NB, _D), lambda i: (i, 0)),
            pl.BlockSpec((_NC, _NB, _H), lambda i: (0, i, 0)),
            pl.BlockSpec((_NC, _NB, _H), lambda i: (0, i, 0)),
            pl.BlockSpec((1, 1, _NB), lambda i: (i, 0, 0)),
            pl.BlockSpec((_D, _H), lambda i: (0, 0)),
            pl.BlockSpec((_H, _H), lambda i: (0, 0)),
            pl.BlockSpec((_H, _H), lambda i: (0, 0)),
        ],
        out_specs=pl.BlockSpec((_M, _H), lambda i: (0, 0)),
        out_shape=jax.ShapeDtypeStruct((_M, _H), _F32),
        scratch_shapes=[pltpu.VMEM((_M, _H), _F32), pltpu.VMEM((_M, _H), _F32)],
        compiler_params=pltpu.CompilerParams(
            dimension_semantics=("arbitrary",)),
    )(x, apart_a, apart_b, mol3, w_ox, w_oh, w_ffn)


def kernel(x, edge_attr, edge_index, mol_ids, W_i, W_h, W_o, W_ffn):
    src = edge_index[0]
    dst = edge_index[1]
    w_ix, w_ie = W_i[:_D], W_i[_D:]
    w_ox, w_oh = W_o[:_D], W_o[_D:]
    zeros = jnp.zeros((_CH, _H), _F32)
    iota2 = jnp.arange(_NP, dtype=jnp.int32).reshape(_NP // _CH, _CH)
    mol3 = mol_ids.reshape(_NG, 1, _NB)
    srch = [src[h * _E2:(h + 1) * _E2].reshape(_NW, _NCH, _CH)
            for h in range(2)]
    dsth = [dst[h * _E2:(h + 1) * _E2].reshape(_NW, _NCH, _CH)
            for h in range(2)]
    eah = [edge_attr[h * _E2:(h + 1) * _E2] for h in range(2)]

    p = _tc_matmul(x, w_ix, 2000)

    ps = [_sc_gather(p, srch[h]) for h in range(2)]
    g = [_tc_stage1(ps[h], eah[h], w_ie, W_h) for h in range(2)]

    for depth in range(2):
        last = depth == 1
        parts = [_sc_scatter(g[h], dsth[h], zeros, iota2) for h in range(2)]
        b = _tc_merge(parts[0], parts[1])
        s = [_sc_gather(b, srch[h]) for h in range(2)]
        g = [_tc_iter(ps[h], eah[h], s[h], g[h], w_ie, W_h, last)
             for h in range(2)]

    aparts = [_sc_scatter(g[h], dsth[h], zeros, iota2) for h in range(2)]
    fp = _tc_final(x, aparts[0], aparts[1], mol3, w_ox, w_oh, W_ffn)
    return fp


# speedup vs baseline: 1.0279x; 1.0279x over previous
import functools

import jax
import jax.numpy as jnp
from jax import lax
from jax.experimental import pallas as pl
from jax.experimental.pallas import tpu as pltpu
from jax.experimental.pallas import tpu_sc as plsc

_N, _E, _D, _DE, _H, _M = 10000, 320000, 128, 16, 128, 512

_NC, _NS = 2, 16
_NW = _NC * _NS
_EPW = _E // _NW
_NP = 10240
_RPS = _NP // _NS
_NBUF = 5

_CH = 80
_NCH = _EPW // _CH
_NGRP = _NCH // _NBUF

_CHS = 40
_NCHS = _EPW // _CHS
_WCHS = _RPS // _CHS

_EB = 2560
_EG = _E // _EB
_NB = 1000
_NG = _N // _NB

_F32 = jnp.float32


def _sc_gather_body(table_hbm, idx3_hbm, out_hbm, idxv, rows, *sems):
    c = lax.axis_index("c")
    s = lax.axis_index("s")
    w = s * _NC + c
    base0 = w * _EPW
    pltpu.sync_copy(idx3_hbm.at[w], idxv)

    def issue_gather(k, b):
        pltpu.async_copy(table_hbm.at[idxv.at[k]], rows.at[b], sems[b])

    def wait_rowbuf(b):
        pltpu.make_async_copy(out_hbm.at[pl.ds(0, _CH)], rows.at[b],
                              sems[b]).wait()

    def wait_store(b):
        pltpu.make_async_copy(rows.at[b], out_hbm.at[pl.ds(0, _CH)],
                              sems[b]).wait()

    for b in range(_NBUF):
        issue_gather(b, b)

    def group(g, carry):
        for b in range(_NBUF):
            k = g * _NBUF + b
            wait_rowbuf(b)
            pltpu.async_copy(rows.at[b],
                             out_hbm.at[pl.ds(base0 + k * _CH, _CH)],
                             sems[b])

            @pl.when(g < _NGRP - 1)
            def _next(b=b, k=k):
                wait_store(b)
                issue_gather(k + _NBUF, b)

        return carry

    lax.fori_loop(0, _NGRP, group, 0)
    for b in range(_NBUF):
        wait_store(b)


_sc_gather = pl.kernel(
    _sc_gather_body,
    out_type=jax.ShapeDtypeStruct((_E, _H), _F32),
    mesh=plsc.VectorSubcoreMesh(core_axis_name="c", subcore_axis_name="s"),
    scratch_types=[
        pltpu.VMEM((_NCH, _CH), jnp.int32),
        pltpu.VMEM((_NBUF, _CH, _H), _F32),
    ] + [pltpu.SemaphoreType.DMA] * _NBUF,
)


def _sc_scatter_body(rows_hbm, idx4_hbm, zeros_hbm, iota2_hbm, part_hbm,
                     acc, idxv, iotav, rows, *sems):
    c = lax.axis_index("c")
    s = lax.axis_index("s")
    w = s * _NC + c
    pltpu.sync_copy(iota2_hbm.at[pl.ds(s * _WCHS, _WCHS)], iotav)

    pltpu.sync_copy(zeros_hbm, rows.at[0])
    zd = [pltpu.async_copy(rows.at[0], acc.at[iotav.at[j]], sems[0])
          for j in range(_WCHS)]
    for d in zd:
        d.wait()
    plsc.subcore_barrier()

    base0 = w * _EPW

    def wait_rowbuf(b):
        pltpu.make_async_copy(rows_hbm.at[pl.ds(0, _CHS)], rows.at[b],
                              sems[b]).wait()

    def wait_add(b):
        pltpu.make_async_copy(rows.at[b], acc.at[iotav.at[0]],
                              sems[b]).wait()

    half = _NCHS // 2
    ngrp_p = half // _NBUF
    for p in range(2):
        pbase = base0 + p * half * _CHS

        def issue_load(k, b, pbase=pbase):
            pltpu.async_copy(rows_hbm.at[pl.ds(pbase + k * _CHS, _CHS)],
                             rows.at[b], sems[b])

        pltpu.sync_copy(idx4_hbm.at[w, p], idxv)
        for b in range(_NBUF):
            issue_load(b, b)

        def group(g, carry, issue_load=issue_load):
            for b in range(_NBUF):
                k = g * _NBUF + b
                wait_rowbuf(b)
                pltpu.async_copy(rows.at[b], acc.at[idxv.at[k]], sems[b],
                                 add=True)

                @pl.when(g < ngrp_p - 1)
                def _next(b=b, k=k):
                    wait_add(b)
                    issue_load(k + _NBUF, b)

            return carry

        lax.fori_loop(0, ngrp_p, group, 0)
        for b in range(_NBUF):
            wait_add(b)
    plsc.subcore_barrier()

    descs = {}
    for j in range(_WCHS):
        b = j % _NBUF
        if j >= _NBUF:
            descs[j - _NBUF].wait()
        pltpu.async_copy(acc.at[iotav.at[j]], rows.at[b], sems[b]).wait()
        descs[j] = pltpu.async_copy(
            rows.at[b], part_hbm.at[c, pl.ds(s * _RPS + j * _CHS, _CHS)],
            sems[b])
    for j in range(_WCHS - _NBUF, _WCHS):
        descs[j].wait()


_sc_scatter = pl.kernel(
    _sc_scatter_body,
    out_type=jax.ShapeDtypeStruct((_NC, _NP, _H), _F32),
    mesh=plsc.VectorSubcoreMesh(core_axis_name="c", subcore_axis_name="s"),
    scratch_types=[
        pltpu.VMEM_SHARED((_NP, _H), _F32),
        pltpu.VMEM((_NCHS // 2, _CHS), jnp.int32),
        pltpu.VMEM((_WCHS, _CHS), jnp.int32),
        pltpu.VMEM((_NBUF, _CHS, _H), _F32),
    ] + [pltpu.SemaphoreType.DMA] * _NBUF,
)


def _mm_body(x_ref, w_ref, o_ref):
    o_ref[...] = jnp.dot(x_ref[...], w_ref[...], preferred_element_type=_F32)


def _tc_matmul(x, w, block):
    n = x.shape[0]
    return pl.pallas_call(
        _mm_body,
        grid=(n // block,),
        in_specs=[
            pl.BlockSpec((block, x.shape[1]), lambda i: (i, 0)),
            pl.BlockSpec(w.shape, lambda i: (0, 0)),
        ],
        out_specs=pl.BlockSpec((block, w.shape[1]), lambda i: (i, 0)),
        out_shape=jax.ShapeDtypeStruct((n, w.shape[1]), _F32),
    )(x, w)


def _merge_body(p_ref, o_ref):
    o_ref[...] = p_ref[0] + p_ref[1]


def _tc_merge(part):
    return pl.pallas_call(
        _merge_body,
        grid=(_NG,),
        in_specs=[pl.BlockSpec((_NC, _NB, _H), lambda i: (0, i, 0))],
        out_specs=pl.BlockSpec((_NB, _H), lambda i: (i, 0)),
        out_shape=jax.ShapeDtypeStruct((_N, _H), _F32),
    )(part)


def _stage1_body(ps_ref, ea_ref, wie_ref, wh_ref, o_ref):
    inp = jnp.maximum(
        ps_ref[...] + jnp.dot(ea_ref[...], wie_ref[...],
                              preferred_element_type=_F32), 0.0)
    o_ref[...] = jnp.dot(inp, wh_ref[...], preferred_element_type=_F32)


def _tc_stage1(ps, ea, w_ie, w_h):
    return pl.pallas_call(
        _stage1_body,
        grid=(_EG,),
        in_specs=[
            pl.BlockSpec((_EB, _H), lambda i: (i, 0)),
            pl.BlockSpec((_EB, _DE), lambda i: (i, 0)),
            pl.BlockSpec((_DE, _H), lambda i: (0, 0)),
            pl.BlockSpec((_H, _H), lambda i: (0, 0)),
        ],
        out_specs=pl.BlockSpec((_EB, _H), lambda i: (i, 0)),
        out_shape=jax.ShapeDtypeStruct((_E, _H), _F32),
    )(ps, ea, w_ie, w_h)


def _pair_swap(g):
    up = jnp.concatenate([g[1:], g[:1]], axis=0)
    dn = jnp.concatenate([g[-1:], g[:-1]], axis=0)
    par = (lax.broadcasted_iota(jnp.int32, g.shape, 0) % 2) == 0
    return jnp.where(par, up, dn)


def _iter_body(ps_ref, ea_ref, s_ref, g_ref, wie_ref, wh_ref, o_ref, *, last):
    inp = jnp.maximum(
        ps_ref[...] + jnp.dot(ea_ref[...], wie_ref[...],
                              preferred_element_type=_F32), 0.0)
    h = jnp.maximum(inp + s_ref[...] - _pair_swap(g_ref[...]), 0.0)
    if last:
        o_ref[...] = h
    else:
        o_ref[...] = jnp.dot(h, wh_ref[...], preferred_element_type=_F32)


def _tc_iter(ps, ea, s, g, w_ie, w_h, last):
    return pl.pallas_call(
        functools.partial(_iter_body, last=last),
        grid=(_EG,),
        in_specs=[
            pl.BlockSpec((_EB, _H), lambda i: (i, 0)),
            pl.BlockSpec((_EB, _DE), lambda i: (i, 0)),
            pl.BlockSpec((_EB, _H), lambda i: (i, 0)),
            pl.BlockSpec((_EB, _H), lambda i: (i, 0)),
            pl.BlockSpec((_DE, _H), lambda i: (0, 0)),
            pl.BlockSpec((_H, _H), lambda i: (0, 0)),
        ],
        out_specs=pl.BlockSpec((_EB, _H), lambda i: (i, 0)),
        out_shape=jax.ShapeDtypeStruct((_E, _H), _F32),
    )(ps, ea, s, g, w_ie, w_h)


def _final_body(x_ref, ap_ref, mol_ref, wox_ref, woh_ref, wffn_ref, o_ref,
                sums, cnts):
    i = pl.program_id(0)

    @pl.when(i == 0)
    def _zero():
        sums[...] = jnp.zeros_like(sums)
        cnts[...] = jnp.zeros_like(cnts)

    a = ap_ref[0] + ap_ref[1]
    atom = jnp.maximum(
        jnp.dot(x_ref[...], wox_ref[...], preferred_element_type=_F32)
        + jnp.dot(a, woh_ref[...], preferred_element_type=_F32), 0.0)
    mol = mol_ref[0]
    mrow = lax.broadcasted_iota(jnp.int32, (_M, _NB), 0)
    oh = (mrow == mol).astype(_F32)
    sums[...] += jnp.dot(oh, atom, preferred_element_type=_F32)
    cnts[...] += jnp.dot(oh, jnp.ones_like(atom), preferred_element_type=_F32)

    @pl.when(i == pl.num_programs(0) - 1)
    def _emit():
        mv = sums[...] / jnp.maximum(cnts[...], 1.0)
        o_ref[...] = jnp.maximum(
            jnp.dot(mv, wffn_ref[...], preferred_element_type=_F32), 0.0)


def _tc_final(x, apart, mol3, w_ox, w_oh, w_ffn):
    return pl.pallas_call(
        _final_body,
        grid=(_NG,),
        in_specs=[
            pl.BlockSpec((_NB, _D), lambda i: (i, 0)),
            pl.BlockSpec((_NC, _NB, _H), lambda i: (0, i, 0)),
            pl.BlockSpec((1, 1, _NB), lambda i: (i, 0, 0)),
            pl.BlockSpec((_D, _H), lambda i: (0, 0)),
            pl.BlockSpec((_H, _H), lambda i: (0, 0)),
            pl.BlockSpec((_H, _H), lambda i: (0, 0)),
        ],
        out_specs=pl.BlockSpec((_M, _H), lambda i: (0, 0)),
        out_shape=jax.ShapeDtypeStruct((_M, _H), _F32),
        scratch_shapes=[pltpu.VMEM((_M, _H), _F32), pltpu.VMEM((_M, _H), _F32)],
        compiler_params=pltpu.CompilerParams(
            dimension_semantics=("arbitrary",)),
    )(x, apart, mol3, w_ox, w_oh, w_ffn)


def kernel(x, edge_attr, edge_index, mol_ids, W_i, W_h, W_o, W_ffn):
    src = edge_index[0]
    dst = edge_index[1]
    w_ix, w_ie = W_i[:_D], W_i[_D:]
    w_ox, w_oh = W_o[:_D], W_o[_D:]
    zeros = jnp.zeros((_CHS, _H), _F32)
    src3 = src.reshape(_NW, _NCH, _CH)
    dst4 = dst.reshape(_NW, 2, _NCHS // 2, _CHS)
    iota2 = jnp.arange(_NP, dtype=jnp.int32).reshape(_NP // _CHS, _CHS)
    mol3 = mol_ids.reshape(_NG, 1, _NB)

    p = _tc_matmul(x, w_ix, 2000)

    ps = _sc_gather(p, src3)
    g = _tc_stage1(ps, edge_attr, w_ie, W_h)

    for depth in range(2):
        last = depth == 1
        bpart = _sc_scatter(g, dst4, zeros, iota2)
        b = _tc_merge(bpart)
        s = _sc_gather(b, src3)
        g = _tc_iter(ps, edge_attr, s, g, w_ie, W_h, last)

    apart = _sc_scatter(g, dst4, zeros, iota2)
    fp = _tc_final(x, apart, mol3, w_ox, w_oh, W_ffn)
    return fp


# speedup vs baseline: 1.1233x; 1.0928x over previous
import functools

import jax
import jax.numpy as jnp
from jax import lax
from jax.experimental import pallas as pl
from jax.experimental.pallas import tpu as pltpu
from jax.experimental.pallas import tpu_sc as plsc

_N, _E, _D, _DE, _H, _M = 10000, 320000, 128, 16, 128, 512

_NC, _NS = 2, 16
_NW = _NC * _NS
_EPW = _E // _NW
_NP = 10240
_RPS = _NP // _NS
_NBUF = 5

_CH = 80
_NCH = _EPW // _CH
_NGRP = _NCH // _NBUF

_CHS = 40
_NCHS = _EPW // _CHS
_WCHS = _RPS // _CHS

_EB = 2560
_EG = _E // _EB
_NB = 1000
_NG = _N // _NB

_F32 = jnp.float32


def _sc_gather_body(table_hbm, idx4_hbm, iota2_hbm, out_hbm,
                    tbl, idxv, iotav, rows, *sems):
    c = lax.axis_index("c")
    s = lax.axis_index("s")
    w = s * _NC + c
    pltpu.sync_copy(iota2_hbm.at[pl.ds(s * _WCHS, _WCHS)], iotav)
    sd = {}
    for t in range(_WCHS):
        b = t % _NBUF
        if t >= _NBUF:
            sd[t - _NBUF].wait()
        pltpu.async_copy(table_hbm.at[pl.ds(s * _RPS + t * _CHS, _CHS)],
                         rows.at[b], sems[b]).wait()
        sd[t] = pltpu.async_copy(rows.at[b], tbl.at[iotav.at[t]], sems[b])
    for t in range(_WCHS - _NBUF, _WCHS):
        sd[t].wait()
    plsc.subcore_barrier()

    base0 = w * _EPW

    def wait_rowbuf(b):
        pltpu.make_async_copy(out_hbm.at[pl.ds(0, _CHS)], rows.at[b],
                              sems[b]).wait()

    def wait_store(b):
        pltpu.make_async_copy(rows.at[b], out_hbm.at[pl.ds(0, _CHS)],
                              sems[b]).wait()

    half = _NCHS // 2
    ngrp_p = half // _NBUF
    for p in range(2):
        pbase = base0 + p * half * _CHS

        def issue_gather(k, b):
            pltpu.async_copy(tbl.at[idxv.at[k]], rows.at[b], sems[b])

        pltpu.sync_copy(idx4_hbm.at[w, p], idxv)
        for b in range(_NBUF):
            issue_gather(b, b)

        def group(g, carry, issue_gather=issue_gather, pbase=pbase):
            for b in range(_NBUF):
                k = g * _NBUF + b
                wait_rowbuf(b)
                pltpu.async_copy(rows.at[b],
                                 out_hbm.at[pl.ds(pbase + k * _CHS, _CHS)],
                                 sems[b])

                @pl.when(g < ngrp_p - 1)
                def _next(b=b, k=k):
                    wait_store(b)
                    issue_gather(k + _NBUF, b)

            return carry

        lax.fori_loop(0, ngrp_p, group, 0)
        for b in range(_NBUF):
            wait_store(b)


_sc_gather = pl.kernel(
    _sc_gather_body,
    out_type=jax.ShapeDtypeStruct((_E, _H), _F32),
    mesh=plsc.VectorSubcoreMesh(core_axis_name="c", subcore_axis_name="s"),
    scratch_types=[
        pltpu.VMEM_SHARED((_NP, _H), _F32),
        pltpu.VMEM((_NCHS // 2, _CHS), jnp.int32),
        pltpu.VMEM((_WCHS, _CHS), jnp.int32),
        pltpu.VMEM((_NBUF, _CHS, _H), _F32),
    ] + [pltpu.SemaphoreType.DMA] * _NBUF,
)


def _sc_scatter_body(rows_hbm, idx4_hbm, zeros_hbm, iota2_hbm, part_hbm,
                     acc, idxv, iotav, rows, *sems):
    c = lax.axis_index("c")
    s = lax.axis_index("s")
    w = s * _NC + c
    pltpu.sync_copy(iota2_hbm.at[pl.ds(s * _WCHS, _WCHS)], iotav)

    pltpu.sync_copy(zeros_hbm, rows.at[0])
    zd = [pltpu.async_copy(rows.at[0], acc.at[iotav.at[j]], sems[0])
          for j in range(_WCHS)]
    for d in zd:
        d.wait()
    plsc.subcore_barrier()

    base0 = w * _EPW

    def wait_rowbuf(b):
        pltpu.make_async_copy(rows_hbm.at[pl.ds(0, _CHS)], rows.at[b],
                              sems[b]).wait()

    def wait_add(b):
        pltpu.make_async_copy(rows.at[b], acc.at[iotav.at[0]],
                              sems[b]).wait()

    half = _NCHS // 2
    ngrp_p = half // _NBUF
    for p in range(2):
        pbase = base0 + p * half * _CHS

        def issue_load(k, b, pbase=pbase):
            pltpu.async_copy(rows_hbm.at[pl.ds(pbase + k * _CHS, _CHS)],
                             rows.at[b], sems[b])

        pltpu.sync_copy(idx4_hbm.at[w, p], idxv)
        for b in range(_NBUF):
            issue_load(b, b)

        def group(g, carry, issue_load=issue_load):
            for b in range(_NBUF):
                k = g * _NBUF + b
                wait_rowbuf(b)
                pltpu.async_copy(rows.at[b], acc.at[idxv.at[k]], sems[b],
                                 add=True)

                @pl.when(g < ngrp_p - 1)
                def _next(b=b, k=k):
                    wait_add(b)
                    issue_load(k + _NBUF, b)

            return carry

        lax.fori_loop(0, ngrp_p, group, 0)
        for b in range(_NBUF):
            wait_add(b)
    plsc.subcore_barrier()

    descs = {}
    for j in range(_WCHS):
        b = j % _NBUF
        if j >= _NBUF:
            descs[j - _NBUF].wait()
        pltpu.async_copy(acc.at[iotav.at[j]], rows.at[b], sems[b]).wait()
        descs[j] = pltpu.async_copy(
            rows.at[b], part_hbm.at[c, pl.ds(s * _RPS + j * _CHS, _CHS)],
            sems[b])
    for j in range(_WCHS - _NBUF, _WCHS):
        descs[j].wait()


_sc_scatter = pl.kernel(
    _sc_scatter_body,
    out_type=jax.ShapeDtypeStruct((_NC, _NP, _H), _F32),
    mesh=plsc.VectorSubcoreMesh(core_axis_name="c", subcore_axis_name="s"),
    scratch_types=[
        pltpu.VMEM_SHARED((_NP, _H), _F32),
        pltpu.VMEM((_NCHS // 2, _CHS), jnp.int32),
        pltpu.VMEM((_WCHS, _CHS), jnp.int32),
        pltpu.VMEM((_NBUF, _CHS, _H), _F32),
    ] + [pltpu.SemaphoreType.DMA] * _NBUF,
)


def _mm_body(x_ref, w_ref, o_ref):
    o_ref[...] = jnp.dot(x_ref[...], w_ref[...], preferred_element_type=_F32)


def _tc_matmul(x, w, block):
    n = x.shape[0]
    return pl.pallas_call(
        _mm_body,
        grid=(n // block,),
        in_specs=[
            pl.BlockSpec((block, x.shape[1]), lambda i: (i, 0)),
            pl.BlockSpec(w.shape, lambda i: (0, 0)),
        ],
        out_specs=pl.BlockSpec((block, w.shape[1]), lambda i: (i, 0)),
        out_shape=jax.ShapeDtypeStruct((_NP, w.shape[1]), _F32),
    )(x, w)


def _merge_body(p_ref, o_ref):
    o_ref[...] = p_ref[0] + p_ref[1]


def _tc_merge(part):
    return pl.pallas_call(
        _merge_body,
        grid=(_NG,),
        in_specs=[pl.BlockSpec((_NC, _NB, _H), lambda i: (0, i, 0))],
        out_specs=pl.BlockSpec((_NB, _H), lambda i: (i, 0)),
        out_shape=jax.ShapeDtypeStruct((_NP, _H), _F32),
    )(part)


def _stage1_body(ps_ref, ea_ref, wie_ref, wh_ref, o_ref):
    inp = jnp.maximum(
        ps_ref[...] + jnp.dot(ea_ref[...], wie_ref[...],
                              preferred_element_type=_F32), 0.0)
    o_ref[...] = jnp.dot(inp, wh_ref[...], preferred_element_type=_F32)


def _tc_stage1(ps, ea, w_ie, w_h):
    return pl.pallas_call(
        _stage1_body,
        grid=(_EG,),
        in_specs=[
            pl.BlockSpec((_EB, _H), lambda i: (i, 0)),
            pl.BlockSpec((_EB, _DE), lambda i: (i, 0)),
            pl.BlockSpec((_DE, _H), lambda i: (0, 0)),
            pl.BlockSpec((_H, _H), lambda i: (0, 0)),
        ],
        out_specs=pl.BlockSpec((_EB, _H), lambda i: (i, 0)),
        out_shape=jax.ShapeDtypeStruct((_E, _H), _F32),
    )(ps, ea, w_ie, w_h)


def _pair_swap(g):
    up = jnp.concatenate([g[1:], g[:1]], axis=0)
    dn = jnp.concatenate([g[-1:], g[:-1]], axis=0)
    par = (lax.broadcasted_iota(jnp.int32, g.shape, 0) % 2) == 0
    return jnp.where(par, up, dn)


def _iter_body(ps_ref, ea_ref, s_ref, g_ref, wie_ref, wh_ref, o_ref, *, last):
    inp = jnp.maximum(
        ps_ref[...] + jnp.dot(ea_ref[...], wie_ref[...],
                              preferred_element_type=_F32), 0.0)
    h = jnp.maximum(inp + s_ref[...] - _pair_swap(g_ref[...]), 0.0)
    if last:
        o_ref[...] = h
    else:
        o_ref[...] = jnp.dot(h, wh_ref[...], preferred_element_type=_F32)


def _tc_iter(ps, ea, s, g, w_ie, w_h, last):
    return pl.pallas_call(
        functools.partial(_iter_body, last=last),
        grid=(_EG,),
        in_specs=[
            pl.BlockSpec((_EB, _H), lambda i: (i, 0)),
            pl.BlockSpec((_EB, _DE), lambda i: (i, 0)),
            pl.BlockSpec((_EB, _H), lambda i: (i, 0)),
            pl.BlockSpec((_EB, _H), lambda i: (i, 0)),
            pl.BlockSpec((_DE, _H), lambda i: (0, 0)),
            pl.BlockSpec((_H, _H), lambda i: (0, 0)),
        ],
        out_specs=pl.BlockSpec((_EB, _H), lambda i: (i, 0)),
        out_shape=jax.ShapeDtypeStruct((_E, _H), _F32),
    )(ps, ea, s, g, w_ie, w_h)


def _final_body(x_ref, ap_ref, mol_ref, wox_ref, woh_ref, wffn_ref, o_ref,
                sums, cnts):
    i = pl.program_id(0)

    @pl.when(i == 0)
    def _zero():
        sums[...] = jnp.zeros_like(sums)
        cnts[...] = jnp.zeros_like(cnts)

    a = ap_ref[0] + ap_ref[1]
    atom = jnp.maximum(
        jnp.dot(x_ref[...], wox_ref[...], preferred_element_type=_F32)
        + jnp.dot(a, woh_ref[...], preferred_element_type=_F32), 0.0)
    mol = mol_ref[0]
    mrow = lax.broadcasted_iota(jnp.int32, (_M, _NB), 0)
    oh = (mrow == mol).astype(_F32)
    sums[...] += jnp.dot(oh, atom, preferred_element_type=_F32)
    cnts[...] += jnp.dot(oh, jnp.ones_like(atom), preferred_element_type=_F32)

    @pl.when(i == pl.num_programs(0) - 1)
    def _emit():
        mv = sums[...] / jnp.maximum(cnts[...], 1.0)
        o_ref[...] = jnp.maximum(
            jnp.dot(mv, wffn_ref[...], preferred_element_type=_F32), 0.0)


def _tc_final(x, apart, mol3, w_ox, w_oh, w_ffn):
    return pl.pallas_call(
        _final_body,
        grid=(_NG,),
        in_specs=[
            pl.BlockSpec((_NB, _D), lambda i: (i, 0)),
            pl.BlockSpec((_NC, _NB, _H), lambda i: (0, i, 0)),
            pl.BlockSpec((1, 1, _NB), lambda i: (i, 0, 0)),
            pl.BlockSpec((_D, _H), lambda i: (0, 0)),
            pl.BlockSpec((_H, _H), lambda i: (0, 0)),
            pl.BlockSpec((_H, _H), lambda i: (0, 0)),
        ],
        out_specs=pl.BlockSpec((_M, _H), lambda i: (0, 0)),
        out_shape=jax.ShapeDtypeStruct((_M, _H), _F32),
        scratch_shapes=[pltpu.VMEM((_M, _H), _F32), pltpu.VMEM((_M, _H), _F32)],
        compiler_params=pltpu.CompilerParams(
            dimension_semantics=("arbitrary",)),
    )(x, apart, mol3, w_ox, w_oh, w_ffn)


def kernel(x, edge_attr, edge_index, mol_ids, W_i, W_h, W_o, W_ffn):
    src = edge_index[0]
    dst = edge_index[1]
    w_ix, w_ie = W_i[:_D], W_i[_D:]
    w_ox, w_oh = W_o[:_D], W_o[_D:]
    zeros = jnp.zeros((_CHS, _H), _F32)
    src4 = src.reshape(_NW, 2, _NCHS // 2, _CHS)
    dst4 = dst.reshape(_NW, 2, _NCHS // 2, _CHS)
    iota2 = jnp.arange(_NP, dtype=jnp.int32).reshape(_NP // _CHS, _CHS)
    mol3 = mol_ids.reshape(_NG, 1, _NB)

    p = _tc_matmul(x, w_ix, 2000)

    ps = _sc_gather(p, src4, iota2)
    g = _tc_stage1(ps, edge_attr, w_ie, W_h)

    for depth in range(2):
        last = depth == 1
        bpart = _sc_scatter(g, dst4, zeros, iota2)
        b = _tc_merge(bpart)
        s = _sc_gather(b, src4, iota2)
        g = _tc_iter(ps, edge_attr, s, g, w_ie, W_h, last)

    apart = _sc_scatter(g, dst4, zeros, iota2)
    fp = _tc_final(x, apart, mol3, w_ox, w_oh, W_ffn)
    return fp
